# baseline (speedup 1.0000x reference)
import functools

import jax
import jax.numpy as jnp
from jax import lax
from jax.experimental import pallas as pl
from jax.experimental.pallas import tpu as pltpu
from jax.experimental.pallas import tpu_sc as plsc

_NUM_CLASSES = 1000
_BATCH = 16384
_NC = 2
_NS = 16
_L = 16
_NW = _NC * _NS
_ROWS_PER_W = _BATCH // _NW
_CHUNK = 16
_NCHUNK = _ROWS_PER_W // _CHUNK
_BUF = _CHUNK * _NUM_CLASSES


def _onehot_body(c_hbm, out_hbm, idx_v, buf0, buf1, sem0, sem1):
    wid = lax.axis_index("s") * _NC + lax.axis_index("c")
    base_row = wid * _ROWS_PER_W

    pltpu.sync_copy(c_hbm.at[pl.ds(base_row, _ROWS_PER_W)], idx_v)

    zeros16 = jnp.zeros((_L,), jnp.float32)
    ones16 = jnp.ones((_L,), jnp.float32)
    iota16 = lax.iota(jnp.int32, _L)

    col_offsets = [j * _L for j in range(_NUM_CLASSES // _L)] + [_NUM_CLASSES - _L]

    def _zero_fill(row, _):
        for off in col_offsets:
            buf0[row, pl.ds(off, _L)] = zeros16
            buf1[row, pl.ds(off, _L)] = zeros16
        return 0

    lax.fori_loop(0, _CHUNK, _zero_fill, 0)

    bufs = (buf0, buf1)
    sems = (sem0, sem1)
    pending = [None, None]

    for chunk in range(_NCHUNK):
        b = chunk % 2
        buf, sem = bufs[b], sems[b]
        if pending[b] is not None:
            pending[b].wait()
            for r in range(_CHUNK // _L):
                col = idx_v[pl.ds((chunk - 2) * _CHUNK + r * _L, _L)]
                rows = iota16 + r * _L
                plsc.store_scatter(buf, [rows, col], zeros16)
        for r in range(_CHUNK // _L):
            col = idx_v[pl.ds(chunk * _CHUNK + r * _L, _L)]
            rows = iota16 + r * _L
            plsc.store_scatter(buf, [rows, col], ones16)
        pending[b] = pltpu.async_copy(
            buf, out_hbm.at[pl.ds(base_row + chunk * _CHUNK, _CHUNK)], sem
        )

    for b in range(2):
        if pending[b] is not None:
            pending[b].wait()


@functools.partial(jax.jit, donate_argnums=())
def kernel(c):
    c = c.astype(jnp.int32)
    mesh = plsc.VectorSubcoreMesh(core_axis_name="c", subcore_axis_name="s")
    run = pl.kernel(
        _onehot_body,
        out_type=jax.ShapeDtypeStruct((_BATCH, _NUM_CLASSES), jnp.float32),
        mesh=mesh,
        scratch_types=[
            pltpu.VMEM((_ROWS_PER_W,), jnp.int32),
            pltpu.VMEM((_CHUNK, _NUM_CLASSES), jnp.float32),
            pltpu.VMEM((_CHUNK, _NUM_CLASSES), jnp.float32),
            pltpu.SemaphoreType.DMA,
            pltpu.SemaphoreType.DMA,
        ],
        compiler_params=pltpu.CompilerParams(
            needs_layout_passes=False, use_tc_tiling_on_sc=False
        ),
    )
    return run(c)


# speedup vs baseline: 1.6203x; 1.6203x over previous
import functools

import jax
import jax.numpy as jnp
from jax import lax
from jax.experimental import pallas as pl
from jax.experimental.pallas import tpu as pltpu
from jax.experimental.pallas import tpu_sc as plsc

_NUM_CLASSES = 1000
_BATCH = 16384
_NC = 2
_NS = 16
_L = 16
_NW = _NC * _NS
_ROWS_PER_W = _BATCH // _NW
_CHUNK = 32
_NCHUNK = _ROWS_PER_W // _CHUNK
_BUF = _CHUNK * _NUM_CLASSES


def _onehot_body(c_hbm, out_hbm, idx_v, buf0, buf1, sem0, sem1):
    wid = lax.axis_index("s") * _NC + lax.axis_index("c")
    base_row = wid * _ROWS_PER_W

    pltpu.sync_copy(c_hbm.at[pl.ds(base_row, _ROWS_PER_W)], idx_v)

    zeros16 = jnp.zeros((_L,), jnp.float32)
    ones16 = jnp.ones((_L,), jnp.float32)
    iota16 = lax.iota(jnp.int32, _L)

    col_offsets = [j * _L for j in range(_NUM_CLASSES // _L)] + [_NUM_CLASSES - _L]

    def _zero_fill(buf):
        def body(row, _):
            for off in col_offsets:
                buf[row, pl.ds(off, _L)] = zeros16
            return 0

        lax.fori_loop(0, _CHUNK, body, 0)

    bufs = (buf0, buf1)
    sems = (sem0, sem1)
    pending = [None, None]

    for chunk in range(_NCHUNK):
        b = chunk % 2
        buf, sem = bufs[b], sems[b]
        if pending[b] is None:
            _zero_fill(buf)
        else:
            pending[b].wait()
            for r in range(_CHUNK // _L):
                col = idx_v[pl.ds((chunk - 2) * _CHUNK + r * _L, _L)]
                rows = iota16 + r * _L
                plsc.store_scatter(buf, [rows, col], zeros16)
        for r in range(_CHUNK // _L):
            col = idx_v[pl.ds(chunk * _CHUNK + r * _L, _L)]
            rows = iota16 + r * _L
            plsc.store_scatter(buf, [rows, col], ones16)
        pending[b] = pltpu.async_copy(
            buf, out_hbm.at[pl.ds(base_row + chunk * _CHUNK, _CHUNK)], sem
        )

    for b in range(2):
        if pending[b] is not None:
            pending[b].wait()


@functools.partial(jax.jit, donate_argnums=())
def kernel(c):
    c = c.astype(jnp.int32)
    mesh = plsc.VectorSubcoreMesh(core_axis_name="c", subcore_axis_name="s")
    run = pl.kernel(
        _onehot_body,
        out_type=jax.ShapeDtypeStruct((_BATCH, _NUM_CLASSES), jnp.float32),
        mesh=mesh,
        scratch_types=[
            pltpu.VMEM((_ROWS_PER_W,), jnp.int32),
            pltpu.VMEM((_CHUNK, _NUM_CLASSES), jnp.float32),
            pltpu.VMEM((_CHUNK, _NUM_CLASSES), jnp.float32),
            pltpu.SemaphoreType.DMA,
            pltpu.SemaphoreType.DMA,
        ],
        compiler_params=pltpu.CompilerParams(needs_layout_passes=False),
    )
    return run(c)


# speedup vs baseline: 3.7435x; 2.3104x over previous
import functools

import jax
import jax.numpy as jnp
from jax import lax
from jax.experimental import pallas as pl
from jax.experimental.pallas import tpu as pltpu
from jax.experimental.pallas import tpu_sc as plsc

_NUM_CLASSES = 1000
_BATCH = 16384
_NC = 2
_NS = 16
_L = 16
_NW = _NC * _NS
_COLS_PER_W = _BATCH // _NW
_CHUNK = 40
_NCHUNK = _NUM_CLASSES // _CHUNK


def _onehot_t_body(c_hbm, out_hbm, idx_v, buf0, buf1, sem0, sem1):
    wid = lax.axis_index("s") * _NC + lax.axis_index("c")
    col0 = wid * _COLS_PER_W

    pltpu.sync_copy(c_hbm.at[pl.ds(col0, _COLS_PER_W)], idx_v)

    zeros16 = jnp.zeros((_L,), jnp.float32)
    ones16 = jnp.ones((_L,), jnp.float32)
    iota16 = lax.iota(jnp.int32, _L)

    def _zero_fill(buf):
        def body(row, _):
            for off in range(0, _COLS_PER_W, _L):
                buf[row, pl.ds(off, _L)] = zeros16
            return 0

        lax.fori_loop(0, _CHUNK, body, 0)

    def _scatter(buf, c0, vals):
        def body(k, _):
            off = pl.multiple_of(k * _L, _L)
            cls = idx_v[pl.ds(off, _L)]
            mask = (cls >= c0) & (cls < c0 + _CHUNK)
            lrow = jnp.where(mask, cls - c0, 0)
            plsc.store_scatter(buf, [lrow, iota16 + k * _L], vals, mask=mask)
            return 0

        lax.fori_loop(0, _COLS_PER_W // _L, body, 0)

    bufs = (buf0, buf1)
    sems = (sem0, sem1)
    pending = [None, None]

    for chunk in range(_NCHUNK):
        b = chunk % 2
        buf, sem = bufs[b], sems[b]
        if pending[b] is None:
            _zero_fill(buf)
        else:
            pending[b].wait()
            _scatter(buf, (chunk - 2) * _CHUNK, zeros16)
        _scatter(buf, chunk * _CHUNK, ones16)
        pending[b] = pltpu.async_copy(
            buf, out_hbm.at[pl.ds(chunk * _CHUNK, _CHUNK), pl.ds(col0, _COLS_PER_W)], sem
        )

    for b in range(2):
        if pending[b] is not None:
            pending[b].wait()


@jax.jit
def kernel(c):
    c = c.astype(jnp.int32)
    mesh = plsc.VectorSubcoreMesh(core_axis_name="c", subcore_axis_name="s")
    run = pl.kernel(
        _onehot_t_body,
        out_type=jax.ShapeDtypeStruct((_NUM_CLASSES, _BATCH), jnp.float32),
        mesh=mesh,
        scratch_types=[
            pltpu.VMEM((_COLS_PER_W,), jnp.int32),
            pltpu.VMEM((_CHUNK, _COLS_PER_W), jnp.float32),
            pltpu.VMEM((_CHUNK, _COLS_PER_W), jnp.float32),
            pltpu.SemaphoreType.DMA,
            pltpu.SemaphoreType.DMA,
        ],
        compiler_params=pltpu.CompilerParams(needs_layout_passes=False),
    )
    return run(c).T
